# Initial kernel scaffold; baseline (speedup 1.0000x reference)
import jax
import jax.numpy as jnp
from jax.experimental import pallas as pl


def kernel(x, edge_index, W1l, b1, W1r, W2l, b2, W2r):
    raise NotImplementedError("write your pallas kernel here")


# speedup vs baseline: 2.9526x; 2.9526x over previous
import jax
import jax.numpy as jnp
from jax import lax
from jax.experimental import pallas as pl
from jax.experimental.pallas import tpu as pltpu
from jax.experimental.pallas import tpu_sc as plsc

_N_SUB = 16
_N_CORE = 2
_CHUNK = 128


def _seg_sum_sc(n_nodes, n_acc, e_pad, d_half, feature_split):
    n_workers = _N_SUB if feature_split else (_N_SUB * _N_CORE)
    per_w = e_pad // n_workers
    n_chunks = per_w // _CHUNK
    rows_out = n_acc // _N_SUB

    mesh = plsc.VectorSubcoreMesh(core_axis_name="c", subcore_axis_name="s")
    out_type = jax.ShapeDtypeStruct((_N_CORE, n_acc, d_half), jnp.float32)
    scratch = [
        pltpu.VMEM((_CHUNK,), jnp.int32),
        pltpu.VMEM((_CHUNK,), jnp.int32),
        pltpu.VMEM((_CHUNK, d_half), jnp.float32),
        pltpu.VMEM_SHARED((n_acc, d_half), jnp.float32),
        pltpu.SemaphoreType.DMA,
    ]

    def body(src_hbm, dst_hbm, table_hbm, zrow_hbm, out_hbm,
             sidx, didx, rows, acc, sem):
        c = lax.axis_index("c")
        s = lax.axis_index("s")
        r0 = s * rows_out

        pltpu.sync_copy(zrow_hbm, acc.at[pl.ds(r0, rows_out)])
        plsc.subcore_barrier()

        if feature_split:
            sbase0 = c * e_pad + s * per_w
            dbase0 = s * per_w
        else:
            sbase0 = dbase0 = (c * _N_SUB + s) * per_w

        def step(i, carry):
            sbase = sbase0 + i * _CHUNK
            dbase = dbase0 + i * _CHUNK
            pltpu.sync_copy(src_hbm.at[pl.ds(sbase, _CHUNK)], sidx)
            pltpu.async_copy(table_hbm.at[sidx], rows, sem).wait()
            pltpu.sync_copy(dst_hbm.at[pl.ds(dbase, _CHUNK)], didx)
            pltpu.sync_copy(rows, acc.at[didx], add=True)
            return carry

        lax.fori_loop(0, n_chunks, step, 0)
        plsc.subcore_barrier()

        pltpu.sync_copy(acc.at[pl.ds(r0, rows_out)],
                        out_hbm.at[c, pl.ds(r0, rows_out)])

    return pl.kernel(body, mesh=mesh, out_type=out_type, scratch_types=scratch)


def _deg_sc(n_acc, e_pad, dw=128):
    per_w = e_pad // (_N_SUB * _N_CORE)
    n_chunks = per_w // _CHUNK
    rows_out = n_acc // _N_SUB

    mesh = plsc.VectorSubcoreMesh(core_axis_name="c", subcore_axis_name="s")

    def body(dst_hbm, zrow_hbm, ones_hbm, out_hbm, didx, ones_v, acc):
        c = lax.axis_index("c")
        s = lax.axis_index("s")
        r0 = s * rows_out
        pltpu.sync_copy(zrow_hbm, acc.at[pl.ds(r0, rows_out)])
        pltpu.sync_copy(ones_hbm, ones_v)
        plsc.subcore_barrier()
        base0 = (c * _N_SUB + s) * per_w

        def step(i, carry):
            pltpu.sync_copy(dst_hbm.at[pl.ds(base0 + i * _CHUNK, _CHUNK)], didx)
            pltpu.sync_copy(ones_v, acc.at[didx], add=True)
            return carry

        lax.fori_loop(0, n_chunks, step, 0)
        plsc.subcore_barrier()
        pltpu.sync_copy(acc.at[pl.ds(r0, rows_out)],
                        out_hbm.at[c, pl.ds(r0, rows_out)])

    return pl.kernel(
        body, mesh=mesh,
        out_type=jax.ShapeDtypeStruct((_N_CORE, n_acc, dw), jnp.float32),
        scratch_types=[
            pltpu.VMEM((_CHUNK,), jnp.int32),
            pltpu.VMEM((_CHUNK, dw), jnp.float32),
            pltpu.VMEM_SHARED((n_acc, dw), jnp.float32),
        ])


def _project1(x, wl, wr, nb=10):
    n, din = x.shape
    dh = wl.shape[1]
    bh = dh // 2
    rb = n // nb

    def bodyf(x_ref, wl_ref, wr_ref, p_ref, xr_ref):
        xb = x_ref[...]
        p_ref[...] = jnp.dot(xb, wl_ref[...], preferred_element_type=jnp.float32)
        xr_ref[...] = jnp.dot(xb, wr_ref[...], preferred_element_type=jnp.float32)

    return pl.pallas_call(
        bodyf,
        grid=(2, nb),
        in_specs=[
            pl.BlockSpec((rb, din), lambda c, b: (b, 0)),
            pl.BlockSpec((din, bh), lambda c, b: (0, c)),
            pl.BlockSpec((din, bh), lambda c, b: (0, c)),
        ],
        out_specs=[
            pl.BlockSpec((rb, bh), lambda c, b: (c * nb + b, 0)),
            pl.BlockSpec((rb, bh), lambda c, b: (b, c)),
        ],
        out_shape=[
            jax.ShapeDtypeStruct((2 * n, bh), jnp.float32),
            jax.ShapeDtypeStruct((n, dh), jnp.float32),
        ],
    )(x, wl, wr)


def _combine_project2(agg_a, agg_b, deg_a, deg_b, xr, b1, w2l, w2r, nb=10):
    n, bh = agg_a.shape
    dh = xr.shape[1]
    do = w2l.shape[1]
    rb = n // nb

    def bodyf(aa, ab, da, db, xrr, b1r, wlr, wrr, p_ref, hr_ref):
        d = jnp.maximum(da[:, :1] + db[:, :1], 1.0)
        aggc = jnp.concatenate([aa[...], ab[...]], axis=1)
        h = jnp.maximum(aggc / d + xrr[...] + b1r[...], 0.0)
        p_ref[...] = jnp.dot(h, wlr[...], preferred_element_type=jnp.float32)
        hr_ref[...] = jnp.dot(h, wrr[...], preferred_element_type=jnp.float32)

    return pl.pallas_call(
        bodyf,
        grid=(nb,),
        in_specs=[
            pl.BlockSpec((rb, bh), lambda b: (b, 0)),
            pl.BlockSpec((rb, bh), lambda b: (b, 0)),
            pl.BlockSpec((rb, deg_a.shape[1]), lambda b: (b, 0)),
            pl.BlockSpec((rb, deg_b.shape[1]), lambda b: (b, 0)),
            pl.BlockSpec((rb, dh), lambda b: (b, 0)),
            pl.BlockSpec((1, dh), lambda b: (0, 0)),
            pl.BlockSpec((dh, do), lambda b: (0, 0)),
            pl.BlockSpec((dh, do), lambda b: (0, 0)),
        ],
        out_specs=[
            pl.BlockSpec((rb, do), lambda b: (b, 0)),
            pl.BlockSpec((rb, do), lambda b: (b, 0)),
        ],
        out_shape=[
            jax.ShapeDtypeStruct((n, do), jnp.float32),
            jax.ShapeDtypeStruct((n, do), jnp.float32),
        ],
    )(agg_a, agg_b, deg_a, deg_b, xr, b1, w2l, w2r)


def _combine_out(a2a, a2b, deg_a, deg_b, hr, b2, nb=10):
    n, do = hr.shape
    rb = n // nb

    def bodyf(aa, ab, da, db, hrr, b2r, o_ref):
        d = jnp.maximum(da[:, :1] + db[:, :1], 1.0)
        o_ref[...] = (aa[...] + ab[...]) / d + hrr[...] + b2r[...]

    return pl.pallas_call(
        bodyf,
        grid=(nb,),
        in_specs=[
            pl.BlockSpec((rb, do), lambda b: (b, 0)),
            pl.BlockSpec((rb, do), lambda b: (b, 0)),
            pl.BlockSpec((rb, deg_a.shape[1]), lambda b: (b, 0)),
            pl.BlockSpec((rb, deg_b.shape[1]), lambda b: (b, 0)),
            pl.BlockSpec((rb, do), lambda b: (b, 0)),
            pl.BlockSpec((1, do), lambda b: (0, 0)),
        ],
        out_specs=pl.BlockSpec((rb, do), lambda b: (b, 0)),
        out_shape=jax.ShapeDtypeStruct((n, do), jnp.float32),
    )(a2a, a2b, deg_a, deg_b, hr, b2)


def kernel(x, edge_index, W1l, b1, W1r, W2l, b2, W2r):
    n, din = x.shape
    e = edge_index.shape[1]
    dh = W1l.shape[1]
    do = W2l.shape[1]

    epc = _N_SUB * _N_CORE * _CHUNK
    e_pad = ((e + epc - 1) // epc) * epc
    n_acc = ((n + 1 + _N_SUB * 8 - 1) // (_N_SUB * 8)) * (_N_SUB * 8)
    rows_out = n_acc // _N_SUB

    src = edge_index[0].astype(jnp.int32)
    dst = edge_index[1].astype(jnp.int32)
    pad = e_pad - e
    src_p = jnp.concatenate([src, jnp.zeros((pad,), jnp.int32)])
    dst_p = jnp.concatenate([dst, jnp.full((pad,), n, jnp.int32)])
    src_stack = jnp.concatenate([src_p, src_p + n])

    zrow1 = jnp.zeros((rows_out, dh // 2), jnp.float32)
    ones_h = jnp.ones((_CHUNK, 128), jnp.float32)

    degf = _deg_sc(n_acc, e_pad)(dst_p, zrow1, ones_h)
    deg_a = degf[0, :n]
    deg_b = degf[1, :n]

    p1, xr = _project1(x, W1l, W1r)
    agg1 = _seg_sum_sc(n, n_acc, e_pad, dh // 2, True)(
        src_stack, dst_p, p1, zrow1)
    a1a = agg1[0, :n]
    a1b = agg1[1, :n]

    p2, hr = _combine_project2(a1a, a1b, deg_a, deg_b, xr,
                               b1.reshape(1, dh), W2l, W2r)
    zrow2 = jnp.zeros((rows_out, do), jnp.float32)
    agg2 = _seg_sum_sc(n, n_acc, e_pad, do, False)(
        src_p, dst_p, p2, zrow2)
    a2a = agg2[0, :n]
    a2b = agg2[1, :n]

    return _combine_out(a2a, a2b, deg_a, deg_b, hr, b2.reshape(1, do))


# speedup vs baseline: 3.7208x; 1.2602x over previous
import jax
import jax.numpy as jnp
from jax import lax
from jax.experimental import pallas as pl
from jax.experimental.pallas import tpu as pltpu
from jax.experimental.pallas import tpu_sc as plsc

_N_SUB = 16
_N_CORE = 2
_CHUNK = 128


def _seg_sum_sc(n_nodes, n_acc, e_pad, d_half, feature_split):
    n_workers = _N_SUB if feature_split else (_N_SUB * _N_CORE)
    per_w = e_pad // n_workers
    n_chunks = per_w // _CHUNK
    n_pairs = n_chunks // 2
    rows_out = n_acc // _N_SUB

    mesh = plsc.VectorSubcoreMesh(core_axis_name="c", subcore_axis_name="s")
    out_type = jax.ShapeDtypeStruct((_N_CORE, n_acc, d_half), jnp.float32)
    scratch = [
        pltpu.VMEM((n_chunks, 1, _CHUNK), jnp.int32),
        pltpu.VMEM((_CHUNK,), jnp.int32),
        pltpu.VMEM((_CHUNK, d_half), jnp.float32),
        pltpu.VMEM((_CHUNK, d_half), jnp.float32),
        pltpu.VMEM_SHARED((n_acc, d_half), jnp.float32),
        pltpu.SemaphoreType.DMA,
        pltpu.SemaphoreType.DMA,
    ]

    def body(src_hbm, dst_hbm, table_hbm, zrow_hbm, out_hbm,
             sidx, didx, rows0, rows1, acc, sem0, sem1):
        c = lax.axis_index("c")
        s = lax.axis_index("s")
        r0 = s * rows_out

        if feature_split:
            srow0 = (c * e_pad) // _CHUNK + s * n_chunks
            dbase0 = s * per_w
        else:
            srow0 = (c * _N_SUB + s) * n_chunks
            dbase0 = (c * _N_SUB + s) * per_w

        pltpu.sync_copy(src_hbm.at[pl.ds(srow0, n_chunks)], sidx)
        pltpu.sync_copy(zrow_hbm, acc.at[pl.ds(r0, rows_out)])
        plsc.subcore_barrier()

        pltpu.async_copy(table_hbm.at[sidx.at[0, 0]], rows0, sem0)

        def step(p, carry):
            j = p * 2
            pltpu.async_copy(table_hbm.at[sidx.at[j + 1, 0]], rows1, sem1)
            pltpu.make_async_copy(table_hbm.at[sidx.at[j, 0]], rows0, sem0).wait()
            pltpu.sync_copy(dst_hbm.at[pl.ds(dbase0 + j * _CHUNK, _CHUNK)], didx)
            pltpu.sync_copy(rows0, acc.at[didx], add=True)
            pltpu.async_copy(table_hbm.at[sidx.at[j + 2, 0]], rows0, sem0)
            pltpu.make_async_copy(table_hbm.at[sidx.at[j + 1, 0]], rows1, sem1).wait()
            pltpu.sync_copy(dst_hbm.at[pl.ds(dbase0 + (j + 1) * _CHUNK, _CHUNK)], didx)
            pltpu.sync_copy(rows1, acc.at[didx], add=True)
            return carry

        lax.fori_loop(0, n_pairs - 1, step, 0)
        j = n_chunks - 2
        pltpu.async_copy(table_hbm.at[sidx.at[j + 1, 0]], rows1, sem1)
        pltpu.make_async_copy(table_hbm.at[sidx.at[j, 0]], rows0, sem0).wait()
        pltpu.sync_copy(dst_hbm.at[pl.ds(dbase0 + j * _CHUNK, _CHUNK)], didx)
        pltpu.sync_copy(rows0, acc.at[didx], add=True)
        pltpu.make_async_copy(table_hbm.at[sidx.at[j + 1, 0]], rows1, sem1).wait()
        pltpu.sync_copy(dst_hbm.at[pl.ds(dbase0 + (j + 1) * _CHUNK, _CHUNK)], didx)
        pltpu.sync_copy(rows1, acc.at[didx], add=True)

        plsc.subcore_barrier()
        pltpu.sync_copy(acc.at[pl.ds(r0, rows_out)],
                        out_hbm.at[c, pl.ds(r0, rows_out)])

    return pl.kernel(body, mesh=mesh, out_type=out_type, scratch_types=scratch)


def _deg_sc(n_acc, e_pad, dw=128):
    per_w = e_pad // (_N_SUB * _N_CORE)
    n_chunks = per_w // _CHUNK
    rows_out = n_acc // _N_SUB

    mesh = plsc.VectorSubcoreMesh(core_axis_name="c", subcore_axis_name="s")

    def body(dst_hbm, zrow_hbm, ones_hbm, out_hbm, didx, ones_v, acc):
        c = lax.axis_index("c")
        s = lax.axis_index("s")
        r0 = s * rows_out
        dbase0 = (c * _N_SUB + s) * per_w
        pltpu.sync_copy(zrow_hbm, acc.at[pl.ds(r0, rows_out)])
        pltpu.sync_copy(ones_hbm, ones_v)
        plsc.subcore_barrier()

        def step(i, carry):
            pltpu.sync_copy(dst_hbm.at[pl.ds(dbase0 + i * _CHUNK, _CHUNK)], didx)
            pltpu.sync_copy(ones_v, acc.at[didx], add=True)
            return carry

        lax.fori_loop(0, n_chunks, step, 0)
        plsc.subcore_barrier()
        pltpu.sync_copy(acc.at[pl.ds(r0, rows_out)],
                        out_hbm.at[c, pl.ds(r0, rows_out)])

    return pl.kernel(
        body, mesh=mesh,
        out_type=jax.ShapeDtypeStruct((_N_CORE, n_acc, dw), jnp.float32),
        scratch_types=[
            pltpu.VMEM((_CHUNK,), jnp.int32),
            pltpu.VMEM((_CHUNK, dw), jnp.float32),
            pltpu.VMEM_SHARED((n_acc, dw), jnp.float32),
        ])


def _project1(x, wl, wr, nb=10):
    n, din = x.shape
    dh = wl.shape[1]
    bh = dh // 2
    rb = n // nb

    def bodyf(x_ref, wl_ref, wr_ref, p_ref, xr_ref):
        xb = x_ref[...]
        p_ref[...] = jnp.dot(xb, wl_ref[...], preferred_element_type=jnp.float32)
        xr_ref[...] = jnp.dot(xb, wr_ref[...], preferred_element_type=jnp.float32)

    return pl.pallas_call(
        bodyf,
        grid=(2, nb),
        in_specs=[
            pl.BlockSpec((rb, din), lambda c, b: (b, 0)),
            pl.BlockSpec((din, bh), lambda c, b: (0, c)),
            pl.BlockSpec((din, bh), lambda c, b: (0, c)),
        ],
        out_specs=[
            pl.BlockSpec((rb, bh), lambda c, b: (c * nb + b, 0)),
            pl.BlockSpec((rb, bh), lambda c, b: (b, c)),
        ],
        out_shape=[
            jax.ShapeDtypeStruct((2 * n, bh), jnp.float32),
            jax.ShapeDtypeStruct((n, dh), jnp.float32),
        ],
    )(x, wl, wr)


def _combine_project2(agg_a, agg_b, deg_a, deg_b, xr, b1, w2l, w2r, nb=10):
    n, bh = agg_a.shape
    dh = xr.shape[1]
    do = w2l.shape[1]
    rb = n // nb

    def bodyf(aa, ab, da, db, xrr, b1r, wlr, wrr, p_ref, hr_ref):
        d = jnp.maximum(da[:, :1] + db[:, :1], 1.0)
        aggc = jnp.concatenate([aa[...], ab[...]], axis=1)
        h = jnp.maximum(aggc / d + xrr[...] + b1r[...], 0.0)
        p_ref[...] = jnp.dot(h, wlr[...], preferred_element_type=jnp.float32)
        hr_ref[...] = jnp.dot(h, wrr[...], preferred_element_type=jnp.float32)

    return pl.pallas_call(
        bodyf,
        grid=(nb,),
        in_specs=[
            pl.BlockSpec((rb, bh), lambda b: (b, 0)),
            pl.BlockSpec((rb, bh), lambda b: (b, 0)),
            pl.BlockSpec((rb, deg_a.shape[1]), lambda b: (b, 0)),
            pl.BlockSpec((rb, deg_b.shape[1]), lambda b: (b, 0)),
            pl.BlockSpec((rb, dh), lambda b: (b, 0)),
            pl.BlockSpec((1, dh), lambda b: (0, 0)),
            pl.BlockSpec((dh, do), lambda b: (0, 0)),
            pl.BlockSpec((dh, do), lambda b: (0, 0)),
        ],
        out_specs=[
            pl.BlockSpec((rb, do), lambda b: (b, 0)),
            pl.BlockSpec((rb, do), lambda b: (b, 0)),
        ],
        out_shape=[
            jax.ShapeDtypeStruct((n, do), jnp.float32),
            jax.ShapeDtypeStruct((n, do), jnp.float32),
        ],
    )(agg_a, agg_b, deg_a, deg_b, xr, b1, w2l, w2r)


def _combine_out(a2a, a2b, deg_a, deg_b, hr, b2, nb=10):
    n, do = hr.shape
    rb = n // nb

    def bodyf(aa, ab, da, db, hrr, b2r, o_ref):
        d = jnp.maximum(da[:, :1] + db[:, :1], 1.0)
        o_ref[...] = (aa[...] + ab[...]) / d + hrr[...] + b2r[...]

    return pl.pallas_call(
        bodyf,
        grid=(nb,),
        in_specs=[
            pl.BlockSpec((rb, do), lambda b: (b, 0)),
            pl.BlockSpec((rb, do), lambda b: (b, 0)),
            pl.BlockSpec((rb, deg_a.shape[1]), lambda b: (b, 0)),
            pl.BlockSpec((rb, deg_b.shape[1]), lambda b: (b, 0)),
            pl.BlockSpec((rb, do), lambda b: (b, 0)),
            pl.BlockSpec((1, do), lambda b: (0, 0)),
        ],
        out_specs=pl.BlockSpec((rb, do), lambda b: (b, 0)),
        out_shape=jax.ShapeDtypeStruct((n, do), jnp.float32),
    )(a2a, a2b, deg_a, deg_b, hr, b2)


def kernel(x, edge_index, W1l, b1, W1r, W2l, b2, W2r):
    n, din = x.shape
    e = edge_index.shape[1]
    dh = W1l.shape[1]
    do = W2l.shape[1]

    epc = _N_SUB * _N_CORE * _CHUNK
    e_pad = ((e + epc - 1) // epc) * epc
    n_acc = ((n + 1 + _N_SUB * 8 - 1) // (_N_SUB * 8)) * (_N_SUB * 8)
    rows_out = n_acc // _N_SUB

    src = edge_index[0].astype(jnp.int32)
    dst = edge_index[1].astype(jnp.int32)
    pad = e_pad - e
    src_p = jnp.concatenate([src, jnp.zeros((pad,), jnp.int32)])
    dst_p = jnp.concatenate([dst, jnp.full((pad,), n, jnp.int32)])
    src_stack = jnp.concatenate([src_p, src_p + n]).reshape(-1, 1, _CHUNK)
    src_p2 = src_p.reshape(-1, 1, _CHUNK)

    zrow1 = jnp.zeros((rows_out, dh // 2), jnp.float32)
    ones_h = jnp.ones((_CHUNK, 128), jnp.float32)

    degf = _deg_sc(n_acc, e_pad)(dst_p, zrow1, ones_h)
    deg_a = degf[0, :n]
    deg_b = degf[1, :n]

    p1, xr = _project1(x, W1l, W1r)
    agg1 = _seg_sum_sc(n, n_acc, e_pad, dh // 2, True)(
        src_stack, dst_p, p1, zrow1)
    a1a = agg1[0, :n]
    a1b = agg1[1, :n]

    p2, hr = _combine_project2(a1a, a1b, deg_a, deg_b, xr,
                               b1.reshape(1, dh), W2l, W2r)
    zrow2 = jnp.zeros((rows_out, do), jnp.float32)
    agg2 = _seg_sum_sc(n, n_acc, e_pad, do, False)(
        src_p2, dst_p, p2, zrow2)
    a2a = agg2[0, :n]
    a2b = agg2[1, :n]

    return _combine_out(a2a, a2b, deg_a, deg_b, hr, b2.reshape(1, do))


# speedup vs baseline: 3.7653x; 1.0120x over previous
import jax
import jax.numpy as jnp
from jax import lax
from jax.experimental import pallas as pl
from jax.experimental.pallas import tpu as pltpu
from jax.experimental.pallas import tpu_sc as plsc

_N_SUB = 16
_N_CORE = 2
_CHUNK = 128
_DEPTH = 2


def _seg_sum_sc(n_nodes, n_acc, e_pad, d_half, feature_split):
    n_workers = _N_SUB if feature_split else (_N_SUB * _N_CORE)
    per_w = e_pad // n_workers
    n_chunks = per_w // _CHUNK
    n_quads = n_chunks // _DEPTH
    rows_out = n_acc // _N_SUB

    mesh = plsc.VectorSubcoreMesh(core_axis_name="c", subcore_axis_name="s")
    out_type = jax.ShapeDtypeStruct((_N_CORE, n_acc, d_half), jnp.float32)
    scratch = (
        [pltpu.VMEM((n_chunks, 1, _CHUNK), jnp.int32)]
        + [pltpu.VMEM((_CHUNK,), jnp.int32) for _ in range(_DEPTH)]
        + [pltpu.VMEM((_CHUNK, d_half), jnp.float32) for _ in range(_DEPTH)]
        + [pltpu.VMEM_SHARED((n_acc, d_half), jnp.float32)]
        + [pltpu.SemaphoreType.DMA for _ in range(2 * _DEPTH)]
    )

    def body(src_hbm, dst_hbm, table_hbm, zrow_hbm, out_hbm, sidx, *rest):
        didxs = rest[:_DEPTH]
        rows = rest[_DEPTH:2 * _DEPTH]
        acc = rest[2 * _DEPTH]
        gsem = rest[2 * _DEPTH + 1:2 * _DEPTH + 1 + _DEPTH]
        dsem = rest[2 * _DEPTH + 1 + _DEPTH:]
        c = lax.axis_index("c")
        s = lax.axis_index("s")
        r0 = s * rows_out

        if feature_split:
            srow0 = (c * e_pad) // _CHUNK + s * n_chunks
            dbase0 = s * per_w
        else:
            srow0 = (c * _N_SUB + s) * n_chunks
            dbase0 = (c * _N_SUB + s) * per_w

        pltpu.sync_copy(src_hbm.at[pl.ds(srow0, n_chunks)], sidx)
        pltpu.sync_copy(zrow_hbm, acc.at[pl.ds(r0, rows_out)])
        plsc.subcore_barrier()

        def issue(jj, k):
            pltpu.async_copy(table_hbm.at[sidx.at[jj, 0]], rows[k], gsem[k])
            pltpu.async_copy(dst_hbm.at[pl.ds(dbase0 + jj * _CHUNK, _CHUNK)],
                             didxs[k], dsem[k])

        def wait_and_scatter(jj, k):
            pltpu.make_async_copy(table_hbm.at[sidx.at[jj, 0]],
                                  rows[k], gsem[k]).wait()
            pltpu.make_async_copy(dst_hbm.at[pl.ds(dbase0 + jj * _CHUNK, _CHUNK)],
                                  didxs[k], dsem[k]).wait()
            pltpu.sync_copy(rows[k], acc.at[didxs[k]], add=True)

        for k in range(_DEPTH):
            issue(k, k)

        def step(q, carry):
            j = q * _DEPTH
            for k in range(_DEPTH):
                wait_and_scatter(j + k, k)
                issue(j + k + _DEPTH, k)
            return carry

        lax.fori_loop(0, n_quads - 1, step, 0)
        j = (n_quads - 1) * _DEPTH
        for k in range(_DEPTH):
            wait_and_scatter(j + k, k)

        plsc.subcore_barrier()
        pltpu.sync_copy(acc.at[pl.ds(r0, rows_out)],
                        out_hbm.at[c, pl.ds(r0, rows_out)])

    return pl.kernel(body, mesh=mesh, out_type=out_type, scratch_types=scratch)


def _deg_sc(n_acc, e_pad, dw=128):
    per_w = e_pad // (_N_SUB * _N_CORE)
    n_chunks = per_w // _CHUNK
    rows_out = n_acc // _N_SUB

    mesh = plsc.VectorSubcoreMesh(core_axis_name="c", subcore_axis_name="s")

    def body(dst_hbm, zrow_hbm, ones_hbm, out_hbm, didx, ones_v, acc):
        c = lax.axis_index("c")
        s = lax.axis_index("s")
        r0 = s * rows_out
        dbase0 = (c * _N_SUB + s) * per_w
        pltpu.sync_copy(zrow_hbm, acc.at[pl.ds(r0, rows_out)])
        pltpu.sync_copy(ones_hbm, ones_v)
        plsc.subcore_barrier()

        def step(i, carry):
            pltpu.sync_copy(dst_hbm.at[pl.ds(dbase0 + i * _CHUNK, _CHUNK)], didx)
            pltpu.sync_copy(ones_v, acc.at[didx], add=True)
            return carry

        lax.fori_loop(0, n_chunks, step, 0)
        plsc.subcore_barrier()
        pltpu.sync_copy(acc.at[pl.ds(r0, rows_out)],
                        out_hbm.at[c, pl.ds(r0, rows_out)])

    return pl.kernel(
        body, mesh=mesh,
        out_type=jax.ShapeDtypeStruct((_N_CORE, n_acc, dw), jnp.float32),
        scratch_types=[
            pltpu.VMEM((_CHUNK,), jnp.int32),
            pltpu.VMEM((_CHUNK, dw), jnp.float32),
            pltpu.VMEM_SHARED((n_acc, dw), jnp.float32),
        ])


def _project1(x, wl, wr, nb=10):
    n, din = x.shape
    dh = wl.shape[1]
    bh = dh // 2
    rb = n // nb

    def bodyf(x_ref, wl_ref, wr_ref, p_ref, xr_ref):
        xb = x_ref[...]
        p_ref[...] = jnp.dot(xb, wl_ref[...], preferred_element_type=jnp.float32)
        xr_ref[...] = jnp.dot(xb, wr_ref[...], preferred_element_type=jnp.float32)

    return pl.pallas_call(
        bodyf,
        grid=(2, nb),
        in_specs=[
            pl.BlockSpec((rb, din), lambda c, b: (b, 0)),
            pl.BlockSpec((din, bh), lambda c, b: (0, c)),
            pl.BlockSpec((din, bh), lambda c, b: (0, c)),
        ],
        out_specs=[
            pl.BlockSpec((rb, bh), lambda c, b: (c * nb + b, 0)),
            pl.BlockSpec((rb, bh), lambda c, b: (b, c)),
        ],
        out_shape=[
            jax.ShapeDtypeStruct((2 * n, bh), jnp.float32),
            jax.ShapeDtypeStruct((n, dh), jnp.float32),
        ],
    )(x, wl, wr)


def _combine_project2(agg_a, agg_b, deg_a, deg_b, xr, b1, w2l, w2r, nb=10):
    n, bh = agg_a.shape
    dh = xr.shape[1]
    do = w2l.shape[1]
    rb = n // nb

    def bodyf(aa, ab, da, db, xrr, b1r, wlr, wrr, p_ref, hr_ref):
        d = jnp.maximum(da[:, :1] + db[:, :1], 1.0)
        aggc = jnp.concatenate([aa[...], ab[...]], axis=1)
        h = jnp.maximum(aggc / d + xrr[...] + b1r[...], 0.0)
        p_ref[...] = jnp.dot(h, wlr[...], preferred_element_type=jnp.float32)
        hr_ref[...] = jnp.dot(h, wrr[...], preferred_element_type=jnp.float32)

    return pl.pallas_call(
        bodyf,
        grid=(nb,),
        in_specs=[
            pl.BlockSpec((rb, bh), lambda b: (b, 0)),
            pl.BlockSpec((rb, bh), lambda b: (b, 0)),
            pl.BlockSpec((rb, deg_a.shape[1]), lambda b: (b, 0)),
            pl.BlockSpec((rb, deg_b.shape[1]), lambda b: (b, 0)),
            pl.BlockSpec((rb, dh), lambda b: (b, 0)),
            pl.BlockSpec((1, dh), lambda b: (0, 0)),
            pl.BlockSpec((dh, do), lambda b: (0, 0)),
            pl.BlockSpec((dh, do), lambda b: (0, 0)),
        ],
        out_specs=[
            pl.BlockSpec((rb, do), lambda b: (b, 0)),
            pl.BlockSpec((rb, do), lambda b: (b, 0)),
        ],
        out_shape=[
            jax.ShapeDtypeStruct((n, do), jnp.float32),
            jax.ShapeDtypeStruct((n, do), jnp.float32),
        ],
    )(agg_a, agg_b, deg_a, deg_b, xr, b1, w2l, w2r)


def _combine_out(a2a, a2b, deg_a, deg_b, hr, b2, nb=10):
    n, do = hr.shape
    rb = n // nb

    def bodyf(aa, ab, da, db, hrr, b2r, o_ref):
        d = jnp.maximum(da[:, :1] + db[:, :1], 1.0)
        o_ref[...] = (aa[...] + ab[...]) / d + hrr[...] + b2r[...]

    return pl.pallas_call(
        bodyf,
        grid=(nb,),
        in_specs=[
            pl.BlockSpec((rb, do), lambda b: (b, 0)),
            pl.BlockSpec((rb, do), lambda b: (b, 0)),
            pl.BlockSpec((rb, deg_a.shape[1]), lambda b: (b, 0)),
            pl.BlockSpec((rb, deg_b.shape[1]), lambda b: (b, 0)),
            pl.BlockSpec((rb, do), lambda b: (b, 0)),
            pl.BlockSpec((1, do), lambda b: (0, 0)),
        ],
        out_specs=pl.BlockSpec((rb, do), lambda b: (b, 0)),
        out_shape=jax.ShapeDtypeStruct((n, do), jnp.float32),
    )(a2a, a2b, deg_a, deg_b, hr, b2)


def kernel(x, edge_index, W1l, b1, W1r, W2l, b2, W2r):
    n, din = x.shape
    e = edge_index.shape[1]
    dh = W1l.shape[1]
    do = W2l.shape[1]

    epc = _N_SUB * _N_CORE * _CHUNK
    e_pad = ((e + epc - 1) // epc) * epc
    n_acc = ((n + 1 + _N_SUB * 8 - 1) // (_N_SUB * 8)) * (_N_SUB * 8)
    rows_out = n_acc // _N_SUB

    src = edge_index[0].astype(jnp.int32)
    dst = edge_index[1].astype(jnp.int32)
    pad = e_pad - e
    src_p = jnp.concatenate([src, jnp.zeros((pad,), jnp.int32)])
    dst_p = jnp.concatenate([dst, jnp.full((pad,), n, jnp.int32)])
    src_stack = jnp.concatenate([src_p, src_p + n]).reshape(-1, 1, _CHUNK)
    src_p2 = src_p.reshape(-1, 1, _CHUNK)

    zrow1 = jnp.zeros((rows_out, dh // 2), jnp.float32)
    ones_h = jnp.ones((_CHUNK, 128), jnp.float32)

    degf = _deg_sc(n_acc, e_pad)(dst_p, zrow1, ones_h)
    deg_a = degf[0, :n]
    deg_b = degf[1, :n]

    p1, xr = _project1(x, W1l, W1r)
    agg1 = _seg_sum_sc(n, n_acc, e_pad, dh // 2, True)(
        src_stack, dst_p, p1, zrow1)
    a1a = agg1[0, :n]
    a1b = agg1[1, :n]

    p2, hr = _combine_project2(a1a, a1b, deg_a, deg_b, xr,
                               b1.reshape(1, dh), W2l, W2r)
    zrow2 = jnp.zeros((rows_out, do), jnp.float32)
    agg2 = _seg_sum_sc(n, n_acc, e_pad, do, False)(
        src_p2, dst_p, p2, zrow2)
    a2a = agg2[0, :n]
    a2b = agg2[1, :n]

    return _combine_out(a2a, a2b, deg_a, deg_b, hr, b2.reshape(1, do))


# speedup vs baseline: 3.8132x; 1.0127x over previous
import jax
import jax.numpy as jnp
from jax import lax
from jax.experimental import pallas as pl
from jax.experimental.pallas import tpu as pltpu
from jax.experimental.pallas import tpu_sc as plsc

_N_SUB = 16
_N_CORE = 2
_CHUNK = 128
_DEPTH = 2


def _seg_sum_sc(n_nodes, n_acc, e_pad, d_half, feature_split):
    n_workers = _N_SUB if feature_split else (_N_SUB * _N_CORE)
    per_w = e_pad // n_workers
    n_chunks = per_w // _CHUNK
    n_quads = n_chunks // _DEPTH
    rows_out = n_acc // _N_SUB

    mesh = plsc.VectorSubcoreMesh(core_axis_name="c", subcore_axis_name="s")
    out_type = jax.ShapeDtypeStruct((_N_CORE, n_acc, d_half), jnp.float32)
    scratch = (
        [pltpu.VMEM((n_chunks, 1, _CHUNK), jnp.int32)]
        + [pltpu.VMEM((_CHUNK,), jnp.int32) for _ in range(_DEPTH)]
        + [pltpu.VMEM((_CHUNK, d_half), jnp.float32) for _ in range(_DEPTH)]
        + [pltpu.VMEM_SHARED((n_acc, d_half), jnp.float32)]
        + [pltpu.SemaphoreType.DMA for _ in range(2 * _DEPTH)]
    )

    def body(src_hbm, dst_hbm, table_hbm, zrow_hbm, out_hbm, sidx, *rest):
        didxs = rest[:_DEPTH]
        rows = rest[_DEPTH:2 * _DEPTH]
        acc = rest[2 * _DEPTH]
        gsem = rest[2 * _DEPTH + 1:2 * _DEPTH + 1 + _DEPTH]
        dsem = rest[2 * _DEPTH + 1 + _DEPTH:]
        c = lax.axis_index("c")
        s = lax.axis_index("s")
        r0 = s * rows_out

        if feature_split:
            srow0 = (c * e_pad) // _CHUNK + s * n_chunks
            dbase0 = s * per_w
        else:
            srow0 = (c * _N_SUB + s) * n_chunks
            dbase0 = (c * _N_SUB + s) * per_w

        pltpu.sync_copy(src_hbm.at[pl.ds(srow0, n_chunks)], sidx)
        pltpu.sync_copy(zrow_hbm, acc.at[pl.ds(r0, rows_out)])
        plsc.subcore_barrier()

        def issue(jj, k):
            pltpu.async_copy(table_hbm.at[sidx.at[jj, 0]], rows[k], gsem[k])
            pltpu.async_copy(dst_hbm.at[pl.ds(dbase0 + jj * _CHUNK, _CHUNK)],
                             didxs[k], dsem[k])

        def wait_and_scatter(jj, k):
            pltpu.make_async_copy(table_hbm.at[sidx.at[jj, 0]],
                                  rows[k], gsem[k]).wait()
            pltpu.make_async_copy(dst_hbm.at[pl.ds(dbase0 + jj * _CHUNK, _CHUNK)],
                                  didxs[k], dsem[k]).wait()
            pltpu.sync_copy(rows[k], acc.at[didxs[k]], add=True)

        for k in range(_DEPTH):
            issue(k, k)

        def step(q, carry):
            j = q * _DEPTH
            for k in range(_DEPTH):
                wait_and_scatter(j + k, k)
                issue(j + k + _DEPTH, k)
            return carry

        lax.fori_loop(0, n_quads - 1, step, 0)
        j = (n_quads - 1) * _DEPTH
        for k in range(_DEPTH):
            wait_and_scatter(j + k, k)

        plsc.subcore_barrier()
        pltpu.sync_copy(acc.at[pl.ds(r0, rows_out)],
                        out_hbm.at[c, pl.ds(r0, rows_out)])

    return pl.kernel(body, mesh=mesh, out_type=out_type, scratch_types=scratch)


def _deg_sc(n_acc, e_pad, dw=128):
    per_w = e_pad // (_N_SUB * _N_CORE)
    n_chunks = per_w // _CHUNK
    rows_out = n_acc // _N_SUB

    mesh = plsc.VectorSubcoreMesh(core_axis_name="c", subcore_axis_name="s")

    def body(dst_hbm, zrow_hbm, ones_hbm, out_hbm, didx, ones_v, acc):
        c = lax.axis_index("c")
        s = lax.axis_index("s")
        r0 = s * rows_out
        dbase0 = (c * _N_SUB + s) * per_w
        pltpu.sync_copy(zrow_hbm, acc.at[pl.ds(r0, rows_out)])
        pltpu.sync_copy(ones_hbm, ones_v)
        plsc.subcore_barrier()

        def step(i, carry):
            pltpu.sync_copy(dst_hbm.at[pl.ds(dbase0 + i * _CHUNK, _CHUNK)], didx)
            pltpu.sync_copy(ones_v, acc.at[didx], add=True)
            return carry

        lax.fori_loop(0, n_chunks, step, 0)
        plsc.subcore_barrier()
        pltpu.sync_copy(acc.at[pl.ds(r0, rows_out)],
                        out_hbm.at[c, pl.ds(r0, rows_out)])

    return pl.kernel(
        body, mesh=mesh,
        out_type=jax.ShapeDtypeStruct((_N_CORE, n_acc, dw), jnp.float32),
        scratch_types=[
            pltpu.VMEM((_CHUNK,), jnp.int32),
            pltpu.VMEM((_CHUNK, dw), jnp.float32),
            pltpu.VMEM_SHARED((n_acc, dw), jnp.float32),
        ])


def _mm_stacked(x, wl, nb=10):
    n, din = x.shape
    dh = wl.shape[1]
    bh = dh // 2
    rb = n // nb

    def bodyf(x_ref, wl_ref, p_ref):
        p_ref[...] = jnp.dot(x_ref[...], wl_ref[...],
                             preferred_element_type=jnp.float32)

    return pl.pallas_call(
        bodyf,
        grid=(2, nb),
        in_specs=[
            pl.BlockSpec((rb, din), lambda c, b: (b, 0)),
            pl.BlockSpec((din, bh), lambda c, b: (0, c)),
        ],
        out_specs=pl.BlockSpec((rb, bh), lambda c, b: (c * nb + b, 0)),
        out_shape=jax.ShapeDtypeStruct((2 * n, bh), jnp.float32),
    )(x, wl)


def _mm(x, w, nb=10):
    n, din = x.shape
    dw = w.shape[1]
    rb = n // nb

    def bodyf(x_ref, w_ref, o_ref):
        o_ref[...] = jnp.dot(x_ref[...], w_ref[...],
                             preferred_element_type=jnp.float32)

    return pl.pallas_call(
        bodyf,
        grid=(nb,),
        in_specs=[
            pl.BlockSpec((rb, din), lambda b: (b, 0)),
            pl.BlockSpec((din, dw), lambda b: (0, 0)),
        ],
        out_specs=pl.BlockSpec((rb, dw), lambda b: (b, 0)),
        out_shape=jax.ShapeDtypeStruct((n, dw), jnp.float32),
    )(x, w)


def _combine_h(agg_a, agg_b, deg_a, deg_b, xr, b1, nb=10):
    n, bh = agg_a.shape
    dh = xr.shape[1]
    rb = n // nb

    def bodyf(aa, ab, da, db, xrr, b1r, h_ref):
        d = jnp.maximum(da[:, :1] + db[:, :1], 1.0)
        aggc = jnp.concatenate([aa[...], ab[...]], axis=1)
        h_ref[...] = jnp.maximum(aggc / d + xrr[...] + b1r[...], 0.0)

    return pl.pallas_call(
        bodyf,
        grid=(nb,),
        in_specs=[
            pl.BlockSpec((rb, bh), lambda b: (b, 0)),
            pl.BlockSpec((rb, bh), lambda b: (b, 0)),
            pl.BlockSpec((rb, deg_a.shape[1]), lambda b: (b, 0)),
            pl.BlockSpec((rb, deg_b.shape[1]), lambda b: (b, 0)),
            pl.BlockSpec((rb, dh), lambda b: (b, 0)),
            pl.BlockSpec((1, dh), lambda b: (0, 0)),
        ],
        out_specs=pl.BlockSpec((rb, dh), lambda b: (b, 0)),
        out_shape=jax.ShapeDtypeStruct((n, dh), jnp.float32),
    )(agg_a, agg_b, deg_a, deg_b, xr, b1)


def _combine_out(a2a, a2b, deg_a, deg_b, hr, b2, nb=10):
    n, do = hr.shape
    rb = n // nb

    def bodyf(aa, ab, da, db, hrr, b2r, o_ref):
        d = jnp.maximum(da[:, :1] + db[:, :1], 1.0)
        o_ref[...] = (aa[...] + ab[...]) / d + hrr[...] + b2r[...]

    return pl.pallas_call(
        bodyf,
        grid=(nb,),
        in_specs=[
            pl.BlockSpec((rb, do), lambda b: (b, 0)),
            pl.BlockSpec((rb, do), lambda b: (b, 0)),
            pl.BlockSpec((rb, deg_a.shape[1]), lambda b: (b, 0)),
            pl.BlockSpec((rb, deg_b.shape[1]), lambda b: (b, 0)),
            pl.BlockSpec((rb, do), lambda b: (b, 0)),
            pl.BlockSpec((1, do), lambda b: (0, 0)),
        ],
        out_specs=pl.BlockSpec((rb, do), lambda b: (b, 0)),
        out_shape=jax.ShapeDtypeStruct((n, do), jnp.float32),
    )(a2a, a2b, deg_a, deg_b, hr, b2)


def kernel(x, edge_index, W1l, b1, W1r, W2l, b2, W2r):
    n, din = x.shape
    e = edge_index.shape[1]
    dh = W1l.shape[1]
    do = W2l.shape[1]

    epc = _N_SUB * _N_CORE * _CHUNK
    e_pad = ((e + epc - 1) // epc) * epc
    n_acc = ((n + 1 + _N_SUB * 8 - 1) // (_N_SUB * 8)) * (_N_SUB * 8)
    rows_out = n_acc // _N_SUB

    src = edge_index[0].astype(jnp.int32)
    dst = edge_index[1].astype(jnp.int32)
    pad = e_pad - e
    src_p = jnp.concatenate([src, jnp.zeros((pad,), jnp.int32)])
    dst_p = jnp.concatenate([dst, jnp.full((pad,), n, jnp.int32)])
    src_stack = jnp.concatenate([src_p, src_p + n]).reshape(-1, 1, _CHUNK)
    src_p2 = src_p.reshape(-1, 1, _CHUNK)

    zrow1 = jnp.zeros((rows_out, dh // 2), jnp.float32)
    ones_h = jnp.ones((_CHUNK, 128), jnp.float32)

    degf = _deg_sc(n_acc, e_pad)(dst_p, zrow1, ones_h)
    deg_a = degf[0, :n]
    deg_b = degf[1, :n]

    p1 = _mm_stacked(x, W1l)
    xr = _mm(x, W1r)
    agg1 = _seg_sum_sc(n, n_acc, e_pad, dh // 2, True)(
        src_stack, dst_p, p1, zrow1)
    a1a = agg1[0, :n]
    a1b = agg1[1, :n]

    h = _combine_h(a1a, a1b, deg_a, deg_b, xr, b1.reshape(1, dh))
    p2 = _mm(h, W2l)
    hr = _mm(h, W2r)
    zrow2 = jnp.zeros((rows_out, do), jnp.float32)
    agg2 = _seg_sum_sc(n, n_acc, e_pad, do, False)(
        src_p2, dst_p, p2, zrow2)
    a2a = agg2[0, :n]
    a2b = agg2[1, :n]

    return _combine_out(a2a, a2b, deg_a, deg_b, hr, b2.reshape(1, do))
